# baseline (speedup 1.0000x reference)
import jax
import jax.numpy as jnp
from jax.experimental import pallas as pl

B, N, C = 32, 8732, 81
NEGPOS_RATIO = 3


def _fused_kernel(x_ref, t_ref, d_ref, o_ref):
    x = x_ref[...]
    t = t_ref[...]
    ce = jnp.maximum(x, 0.0) - x * t + jnp.log1p(jnp.exp(-jnp.abs(x)))
    v = jnp.swapaxes(jnp.max(ce, axis=2, keepdims=True), 1, 2)[:, 0, :]
    d = d_ref[:, 0, :] > 0
    v = jnp.where(d, 0.0, v)
    num_pos = jnp.sum(d.astype(jnp.int32), axis=1, keepdims=True)
    k = jnp.minimum(NEGPOS_RATIO * num_pos, N - 1)
    u = jax.lax.bitcast_convert_type(v, jnp.int32)
    thr = jnp.zeros((1, 1), jnp.int32)
    for b in range(30, -1, -1):
        cand = thr | (1 << b)
        cnt = jnp.sum((u >= cand).astype(jnp.int32), axis=1, keepdims=True)
        thr = jnp.where(cnt >= k, cand, thr)
    m = jnp.sum((u > thr).astype(jnp.int32), axis=1, keepdims=True)
    r = k - m
    eq = u == thr
    idx = jax.lax.broadcasted_iota(jnp.int32, (1, N), 1)
    c = jnp.zeros((1, 1), jnp.int32)
    for b in range(13, -1, -1):
        cand = c | (1 << b)
        cnt = jnp.sum((eq & (idx < cand)).astype(jnp.int32), axis=1,
                      keepdims=True)
        c = jnp.where(cnt <= r, cand, c)
    keep = d | (u > thr) | (eq & (idx < c))
    keep_col = jnp.swapaxes(keep.astype(jnp.float32)[:, None, :], 1, 2)
    o_ref[...] = ce * keep_col


@jax.jit
def kernel(pred_logits, target, depth):
    return pl.pallas_call(
        _fused_kernel,
        grid=(B,),
        in_specs=[
            pl.BlockSpec((1, N, C), lambda i: (i, 0, 0)),
            pl.BlockSpec((1, N, C), lambda i: (i, 0, 0)),
            pl.BlockSpec((1, 1, N), lambda i: (i, 0, 0)),
        ],
        out_specs=pl.BlockSpec((1, N, C), lambda i: (i, 0, 0)),
        out_shape=jax.ShapeDtypeStruct((B, N, C), jnp.float32),
    )(pred_logits, target, depth.reshape(B, 1, N))


# speedup vs baseline: 1.4353x; 1.4353x over previous
import jax
import jax.numpy as jnp
from jax.experimental import pallas as pl
from jax.experimental.pallas import tpu as pltpu

B, N, C = 32, 8732, 81
NEGPOS_RATIO = 3


def _ce_max_kernel(x_ref, t_ref, ce_ref, v_ref):
    x = x_ref[...]
    t = t_ref[...]
    ce = jnp.maximum(x, 0.0) - x * t + jnp.log1p(jnp.exp(-jnp.abs(x)))
    ce_ref[...] = ce
    v_ref[...] = jnp.swapaxes(jnp.max(ce, axis=2, keepdims=True), 1, 2)


def _mask_kernel(v_ref, d_ref, m_ref, z_ref):
    d = d_ref[...] > 0
    v = jnp.where(d, 0.0, v_ref[...])
    num_pos = jnp.sum(d.astype(jnp.int32), axis=1, keepdims=True)
    k = jnp.minimum(NEGPOS_RATIO * num_pos, N - 1)
    u = jax.lax.bitcast_convert_type(v, jnp.int32)
    t = jnp.zeros((B, 1), jnp.int32)
    for b in range(30, -1, -1):
        cand = t | (1 << b)
        cnt = jnp.sum((u >= cand).astype(jnp.int32), axis=1, keepdims=True)
        t = jnp.where(cnt >= k, cand, t)
    m = jnp.sum((u > t).astype(jnp.int32), axis=1, keepdims=True)
    r = k - m
    eq = u == t
    idx = jax.lax.broadcasted_iota(jnp.int32, (B, N), 1)
    c = jnp.zeros((B, 1), jnp.int32)
    for b in range(13, -1, -1):
        cand = c | (1 << b)
        cnt = jnp.sum((eq & (idx < cand)).astype(jnp.int32), axis=1,
                      keepdims=True)
        c = jnp.where(cnt <= r, cand, c)
    keep = d | (u > t) | (eq & (idx < c))
    m_ref[...] = jnp.swapaxes(keep.astype(jnp.float32), 0, 1)
    z_ref[...] = N - jnp.sum(keep.astype(jnp.int32), axis=1, keepdims=True)


def _fixup_kernel(ce_ref, mt_ref, z_ref, o_ref, scratch, sem):
    for b in range(B):
        @pl.when(z_ref[b] > 0)
        def _():
            cp_in = pltpu.make_async_copy(ce_ref.at[b], scratch, sem)
            cp_in.start()
            cp_in.wait()
            scratch[...] = scratch[...] * mt_ref[:, b:b + 1]
            cp_out = pltpu.make_async_copy(scratch, o_ref.at[b], sem)
            cp_out.start()
            cp_out.wait()


@jax.jit
def kernel(pred_logits, target, depth):
    ce, v = pl.pallas_call(
        _ce_max_kernel,
        grid=(B,),
        in_specs=[
            pl.BlockSpec((1, N, C), lambda i: (i, 0, 0)),
            pl.BlockSpec((1, N, C), lambda i: (i, 0, 0)),
        ],
        out_specs=[
            pl.BlockSpec((1, N, C), lambda i: (i, 0, 0)),
            pl.BlockSpec((1, 1, N), lambda i: (i, 0, 0)),
        ],
        out_shape=[
            jax.ShapeDtypeStruct((B, N, C), jnp.float32),
            jax.ShapeDtypeStruct((B, 1, N), jnp.float32),
        ],
    )(pred_logits, target)

    mask, zcnt = pl.pallas_call(
        _mask_kernel,
        out_shape=[
            jax.ShapeDtypeStruct((N, B), jnp.float32),
            jax.ShapeDtypeStruct((B, 1), jnp.int32),
        ],
    )(v.reshape(B, N), depth.reshape(B, N))

    out = pl.pallas_call(
        _fixup_kernel,
        in_specs=[
            pl.BlockSpec(memory_space=pl.ANY),
            pl.BlockSpec(memory_space=pltpu.VMEM),
            pl.BlockSpec(memory_space=pltpu.SMEM),
        ],
        out_specs=pl.BlockSpec(memory_space=pl.ANY),
        out_shape=jax.ShapeDtypeStruct((B, N, C), jnp.float32),
        scratch_shapes=[
            pltpu.VMEM((N, C), jnp.float32),
            pltpu.SemaphoreType.DMA,
        ],
        input_output_aliases={0: 0},
    )(ce, mask, zcnt.reshape(B))

    return out


# speedup vs baseline: 1.4675x; 1.0224x over previous
import jax
import jax.numpy as jnp
from jax.experimental import pallas as pl
from jax.experimental.pallas import tpu as pltpu

B, N, C = 32, 8732, 81
NEGPOS_RATIO = 3


def _ce_max_kernel(x_ref, t_ref, ce_ref, v_ref):
    x = x_ref[...]
    t = t_ref[...]
    ce = jnp.maximum(x, 0.0) - x * t + jnp.log1p(jnp.exp(-jnp.abs(x)))
    ce_ref[...] = ce
    v_ref[...] = jnp.swapaxes(jnp.max(ce, axis=2, keepdims=True), 1, 2)


def _mask_fixup_kernel(v_ref, d_ref, ce_ref, o_ref, scratch, sem):
    d = d_ref[...] > 0
    v = jnp.where(d, 0.0, v_ref[...])
    num_pos = jnp.sum(d.astype(jnp.int32), axis=1, keepdims=True)
    k = jnp.minimum(NEGPOS_RATIO * num_pos, N - 1)
    u = jax.lax.bitcast_convert_type(v, jnp.int32)
    t = jnp.zeros((B, 1), jnp.int32)
    for b in range(30, -1, -1):
        cand = t | (1 << b)
        cnt = jnp.sum((u >= cand).astype(jnp.int32), axis=1, keepdims=True)
        t = jnp.where(cnt >= k, cand, t)
    m = jnp.sum((u > t).astype(jnp.int32), axis=1, keepdims=True)
    r = k - m
    eq = u == t
    idx = jax.lax.broadcasted_iota(jnp.int32, (B, N), 1)
    c = jnp.zeros((B, 1), jnp.int32)
    for b in range(13, -1, -1):
        cand = c | (1 << b)
        cnt = jnp.sum((eq & (idx < cand)).astype(jnp.int32), axis=1,
                      keepdims=True)
        c = jnp.where(cnt <= r, cand, c)
    keep = d | (u > t) | (eq & (idx < c))
    mt = jnp.swapaxes(keep.astype(jnp.float32), 0, 1)
    zcnt = N - jnp.sum(keep.astype(jnp.int32), axis=1, keepdims=True)
    for b in range(B):
        @pl.when(zcnt[b, 0] > 0)
        def _():
            cp_in = pltpu.make_async_copy(ce_ref.at[b], scratch, sem)
            cp_in.start()
            cp_in.wait()
            scratch[...] = scratch[...] * mt[:, b:b + 1]
            cp_out = pltpu.make_async_copy(scratch, o_ref.at[b], sem)
            cp_out.start()
            cp_out.wait()


@jax.jit
def kernel(pred_logits, target, depth):
    ce, v = pl.pallas_call(
        _ce_max_kernel,
        grid=(B,),
        in_specs=[
            pl.BlockSpec((1, N, C), lambda i: (i, 0, 0)),
            pl.BlockSpec((1, N, C), lambda i: (i, 0, 0)),
        ],
        out_specs=[
            pl.BlockSpec((1, N, C), lambda i: (i, 0, 0)),
            pl.BlockSpec((1, 1, N), lambda i: (i, 0, 0)),
        ],
        out_shape=[
            jax.ShapeDtypeStruct((B, N, C), jnp.float32),
            jax.ShapeDtypeStruct((B, 1, N), jnp.float32),
        ],
    )(pred_logits, target)

    out = pl.pallas_call(
        _mask_fixup_kernel,
        in_specs=[
            pl.BlockSpec(memory_space=pltpu.VMEM),
            pl.BlockSpec(memory_space=pltpu.VMEM),
            pl.BlockSpec(memory_space=pl.ANY),
        ],
        out_specs=pl.BlockSpec(memory_space=pl.ANY),
        out_shape=jax.ShapeDtypeStruct((B, N, C), jnp.float32),
        scratch_shapes=[
            pltpu.VMEM((N, C), jnp.float32),
            pltpu.SemaphoreType.DMA,
        ],
        input_output_aliases={2: 0},
    )(v.reshape(B, N), depth.reshape(B, N), ce)

    return out
